# baseline (speedup 1.0000x reference)
import functools

import jax
import jax.numpy as jnp
from jax import lax
from jax.experimental import pallas as pl
from jax.experimental.pallas import tpu as pltpu
from jax.experimental.pallas import tpu_sc as plsc

EMB = 16
BATCH = 16384
VOC = 1000001

_info = plsc.get_sparse_core_info()
_NC, _NS = _info.num_cores, _info.num_subcores
_NW = _NC * _NS
_B_PER_W = BATCH // _NW

_WIN = 32768
_CH = 2048
_N_FULL = 16
_TAIL_LO = 30 * _WIN + 8 * _CH
_TAIL_W = VOC - _TAIL_LO
_CAP = BATCH
_STAGE = 64
_FLUSH_AT = 48

_mesh = plsc.VectorSubcoreMesh(core_axis_name="c", subcore_axis_name="s")


@functools.partial(
    pl.kernel,
    mesh=_mesh,
    out_type=jax.ShapeDtypeStruct((BATCH + _STAGE, 128), jnp.float32),
    compiler_params=pltpu.CompilerParams(needs_layout_passes=False),
    scratch_types=[
        pltpu.VMEM((BATCH,), jnp.int32),
        pltpu.VMEM((_CAP + 16,), jnp.int32),
        pltpu.VMEM((2, EMB, _CH), jnp.float32),
        pltpu.VMEM((32,), jnp.int32),
        pltpu.VMEM((32,), jnp.int32),
        pltpu.VMEM((_STAGE, 128), jnp.float32),
        pltpu.VMEM((_STAGE,), jnp.int32),
        pltpu.SemaphoreType.DMA,
        pltpu.SemaphoreType.DMA,
    ],
)
def _sweep(
    table_hbm,
    idx_hbm,
    acc_hbm,
    idx_v,
    mb_v,
    chunk_v,
    tc_v,
    tb_v,
    stage_v,
    bidx_v,
    sem0,
    sem1,
):
    wid = lax.axis_index("s") * _NC + lax.axis_index("c")
    win_lo = wid * _WIN
    win_hi = jnp.minimum(win_lo + _WIN, VOC)
    lane = lax.iota(jnp.int32, 16)
    sems = (sem0, sem1)
    n_full = jnp.where(wid == 30, 8, jnp.where(wid == 31, 0, _N_FULL))

    pltpu.sync_copy(idx_hbm, idx_v)

    def reset_bidx():
        for q in range(_STAGE // 16):
            bidx_v[pl.ds(16 * q, 16)] = BATCH + 16 * q + lane

    reset_bidx()

    def issue(ch, buf):
        lo = pl.multiple_of(win_lo + ch * _CH, 128)
        pltpu.async_copy(
            table_hbm.at[:, pl.ds(lo, _CH)], chunk_v.at[buf], sems[buf]
        )

    @pl.when(n_full > 0)
    def _():
        issue(0, 0)

    def scan_body(i, cnt):
        for q in range(4):
            ii = i * 4 + q
            c = idx_v[pl.ds(ii * 16, 16)]
            b = jnp.broadcast_to(ii * 16, (16,)) + lane
            m = (c >= win_lo) & (c < win_hi)
            n = plsc.all_reduce_population_count(m)[0]
            plsc.store_compressed(mb_v.at[pl.ds(cnt, 16)], b, mask=m)
            cnt = cnt + n
        return cnt

    cnt = lax.fori_loop(0, BATCH // 64, scan_body, 0)
    mb_v[pl.ds(cnt, 16)] = jnp.broadcast_to(jnp.int32(0), (16,))
    n_vregs = (cnt + 15) >> 4

    def flush(slot):
        pltpu.sync_copy(stage_v, acc_hbm.at[bidx_v])
        reset_bidx()
        return 0

    def extract_matches(lo, buf, m2, bv, cv, slot):
        plsc.store_compressed(tc_v.at[pl.ds(0, 16)], cv, mask=m2)
        plsc.store_compressed(tb_v.at[pl.ds(0, 16)], bv, mask=m2)
        nn = plsc.all_reduce_population_count(m2)[0]

        def match_body(mm, slot):
            cm = tc_v[pl.ds(mm, 16)][0]
            bm = tb_v[pl.ds(mm, 16)][0]
            col = plsc.load_gather(
                chunk_v.at[buf], [lane, jnp.broadcast_to(cm - lo, (16,))]
            )
            plsc.store_scatter(
                stage_v, [jnp.broadcast_to(slot, (16,)), lane], col
            )
            plsc.store_scatter(
                bidx_v,
                [jnp.broadcast_to(slot, (16,))],
                jnp.broadcast_to(bm, (16,)),
                mask=lane == 0,
            )
            slot = slot + 1
            return lax.cond(slot >= _FLUSH_AT, flush, lambda s: s, slot)

        return lax.fori_loop(0, nn, match_body, slot)

    def scan_chunk(lo, width, buf, slot_in):
        def vreg_body(v, slot):
            bv = mb_v[pl.ds(v * 16, 16)]
            cv = plsc.load_gather(idx_v, [bv])
            m2 = (cv - lo >= 0) & (cv - lo < width)
            nn = plsc.all_reduce_population_count(m2)[0]
            return lax.cond(
                nn > 0,
                lambda s: extract_matches(lo, buf, m2, bv, cv, s),
                lambda s: s,
                slot,
            )

        return lax.fori_loop(0, n_vregs, vreg_body, slot_in)

    def process_chunk(ch, buf, slot_in):
        pltpu.make_async_copy(
            table_hbm.at[:, pl.ds(0, _CH)], chunk_v.at[buf], sems[buf]
        ).wait()
        return scan_chunk(win_lo + ch * _CH, _CH, buf, slot_in)

    def chunk_loop(ch, slot):
        @pl.when(ch + 1 < n_full)
        def _():
            @pl.when((ch + 1) % 2 == 0)
            def _():
                issue(ch + 1, 0)

            @pl.when((ch + 1) % 2 == 1)
            def _():
                issue(ch + 1, 1)

        return lax.cond(
            ch % 2 == 0,
            lambda s: process_chunk(ch, 0, s),
            lambda s: process_chunk(ch, 1, s),
            slot,
        )

    slot = lax.fori_loop(0, n_full, chunk_loop, 0)

    @pl.when(wid == 30)
    def _():
        tail_lo = pl.multiple_of(_TAIL_LO + 0 * wid, 128)
        pltpu.sync_copy(
            table_hbm.at[:, pl.ds(tail_lo, 640)],
            chunk_v.at[0, :, pl.ds(0, 640)],
        )

    def tail_fn(slot):
        return scan_chunk(_TAIL_LO, _TAIL_W, 0, slot)

    slot = lax.cond(wid == 30, tail_fn, lambda s: s, slot)
    lax.cond(slot > 0, flush, lambda s: s, slot)


@functools.partial(
    pl.kernel,
    mesh=_mesh,
    out_type=jax.ShapeDtypeStruct((EMB, BATCH), jnp.float32),
    compiler_params=pltpu.CompilerParams(needs_layout_passes=False),
    scratch_types=[
        pltpu.VMEM((_B_PER_W, 128), jnp.float32),
        pltpu.VMEM((EMB, _B_PER_W), jnp.float32),
    ],
)
def _compact(acc_hbm, out_hbm, rb_v, cols_v):
    wid = lax.axis_index("s") * _NC + lax.axis_index("c")
    base = wid * _B_PER_W
    lane = lax.iota(jnp.int32, 16)
    pltpu.sync_copy(acc_hbm.at[pl.ds(base, _B_PER_W)], rb_v)

    def body(u, carry):
        row = plsc.load_gather(rb_v, [jnp.broadcast_to(u, (16,)), lane])
        plsc.store_scatter(cols_v, [lane, jnp.broadcast_to(u, (16,))], row)
        return carry

    lax.fori_loop(0, _B_PER_W, body, 0)
    pltpu.sync_copy(cols_v, out_hbm.at[:, pl.ds(base, _B_PER_W)])


def kernel(user_id, table):
    idx = user_id.astype(jnp.int32)
    acc = _sweep(table.T, idx)
    out_t = _compact(acc)
    return out_t.T[:, None, :]


# speedup vs baseline: 1.0129x; 1.0129x over previous
import functools

import jax
import jax.numpy as jnp
from jax import lax
from jax.experimental import pallas as pl
from jax.experimental.pallas import tpu as pltpu
from jax.experimental.pallas import tpu_sc as plsc

EMB = 16
BATCH = 16384
VOC = 1000001

_info = plsc.get_sparse_core_info()
_NC, _NS = _info.num_cores, _info.num_subcores
_NW = _NC * _NS
_B_PER_W = BATCH // _NW

_WIN = 32768
_CH = 2048
_N_FULL = 16
_TAIL_LO = 30 * _WIN + 8 * _CH
_TAIL_W = VOC - _TAIL_LO
_CAP = BATCH
_STAGE = 64
_FLUSH_AT = 48

_mesh = plsc.VectorSubcoreMesh(core_axis_name="c", subcore_axis_name="s")


@functools.partial(
    pl.kernel,
    mesh=_mesh,
    out_type=jax.ShapeDtypeStruct((BATCH + _STAGE, 128), jnp.float32),
    compiler_params=pltpu.CompilerParams(needs_layout_passes=False),
    scratch_types=[
        pltpu.VMEM((BATCH,), jnp.int32),
        pltpu.VMEM((_CAP + 16,), jnp.int32),
        pltpu.VMEM((2, EMB, _CH), jnp.float32),
        pltpu.VMEM((32,), jnp.int32),
        pltpu.VMEM((32,), jnp.int32),
        pltpu.VMEM((_STAGE, 128), jnp.float32),
        pltpu.VMEM((_STAGE,), jnp.int32),
        pltpu.SemaphoreType.DMA,
        pltpu.SemaphoreType.DMA,
    ],
)
def _sweep(
    table_hbm,
    idx_hbm,
    acc_hbm,
    idx_v,
    mb_v,
    chunk_v,
    tc_v,
    tb_v,
    stage_v,
    bidx_v,
    sem0,
    sem1,
):
    wid = lax.axis_index("s") * _NC + lax.axis_index("c")
    win_lo = wid * _WIN
    win_hi = jnp.minimum(win_lo + _WIN, VOC)
    lane = lax.iota(jnp.int32, 16)
    sems = (sem0, sem1)
    n_full = jnp.where(wid == 30, 8, jnp.where(wid == 31, 0, _N_FULL))

    pltpu.sync_copy(idx_hbm, idx_v)

    def reset_bidx():
        for q in range(_STAGE // 16):
            bidx_v[pl.ds(16 * q, 16)] = BATCH + 16 * q + lane

    reset_bidx()

    def issue(ch, buf):
        lo = pl.multiple_of(win_lo + ch * _CH, 128)
        pltpu.async_copy(
            table_hbm.at[:, pl.ds(lo, _CH)], chunk_v.at[buf], sems[buf]
        )

    @pl.when(n_full > 0)
    def _():
        issue(0, 0)

    def scan_body(i, cnt):
        for q in range(4):
            ii = i * 4 + q
            c = idx_v[pl.ds(ii * 16, 16)]
            b = jnp.broadcast_to(ii * 16, (16,)) + lane
            m = (c - win_lo).astype(jnp.uint32) < (win_hi - win_lo).astype(
                jnp.uint32
            )
            n = plsc.all_reduce_population_count(m)[0]
            plsc.store_compressed(mb_v.at[pl.ds(cnt, 16)], b, mask=m)
            cnt = cnt + n
        return cnt

    cnt = lax.fori_loop(0, BATCH // 64, scan_body, 0)
    mb_v[pl.ds(cnt, 16)] = jnp.broadcast_to(jnp.int32(0), (16,))
    n_vregs = (cnt + 15) >> 4

    def flush(slot):
        pltpu.sync_copy(stage_v, acc_hbm.at[bidx_v])
        reset_bidx()
        return 0

    def scan_chunk(lo, width, buf, slot_in):
        def vreg_body(v, slot):
            bv = mb_v[pl.ds(v * 16, 16)]
            cv = plsc.load_gather(idx_v, [bv])
            m2 = (cv - lo).astype(jnp.uint32) < width
            nn = plsc.all_reduce_population_count(m2)[0]
            plsc.store_compressed(tc_v.at[pl.ds(0, 16)], cv, mask=m2)
            plsc.store_compressed(tb_v.at[pl.ds(0, 16)], bv, mask=m2)
            slot = lax.cond(slot >= _FLUSH_AT, flush, lambda s: s, slot)

            def match_body(mm, slot):
                cm = tc_v[pl.ds(mm, 16)][0]
                bm = tb_v[pl.ds(mm, 16)][0]
                col = plsc.load_gather(
                    chunk_v.at[buf], [lane, jnp.broadcast_to(cm - lo, (16,))]
                )
                plsc.store_scatter(
                    stage_v, [jnp.broadcast_to(slot, (16,)), lane], col
                )
                plsc.store_scatter(
                    bidx_v,
                    [jnp.broadcast_to(slot, (16,))],
                    jnp.broadcast_to(bm, (16,)),
                    mask=lane == 0,
                )
                return slot + 1

            return lax.fori_loop(0, nn, match_body, slot)

        return lax.fori_loop(0, n_vregs, vreg_body, slot_in)

    def process_chunk(ch, buf, slot_in):
        pltpu.make_async_copy(
            table_hbm.at[:, pl.ds(0, _CH)], chunk_v.at[buf], sems[buf]
        ).wait()
        return scan_chunk(win_lo + ch * _CH, _CH, buf, slot_in)

    def chunk_loop(ch, slot):
        @pl.when(ch + 1 < n_full)
        def _():
            @pl.when((ch + 1) % 2 == 0)
            def _():
                issue(ch + 1, 0)

            @pl.when((ch + 1) % 2 == 1)
            def _():
                issue(ch + 1, 1)

        return lax.cond(
            ch % 2 == 0,
            lambda s: process_chunk(ch, 0, s),
            lambda s: process_chunk(ch, 1, s),
            slot,
        )

    slot = lax.fori_loop(0, n_full, chunk_loop, 0)

    @pl.when(wid == 30)
    def _():
        tail_lo = pl.multiple_of(_TAIL_LO + 0 * wid, 128)
        pltpu.sync_copy(
            table_hbm.at[:, pl.ds(tail_lo, 640)],
            chunk_v.at[0, :, pl.ds(0, 640)],
        )

    def tail_fn(slot):
        return scan_chunk(_TAIL_LO, _TAIL_W, 0, slot)

    slot = lax.cond(wid == 30, tail_fn, lambda s: s, slot)
    lax.cond(slot > 0, flush, lambda s: s, slot)


@functools.partial(
    pl.kernel,
    mesh=_mesh,
    out_type=jax.ShapeDtypeStruct((EMB, BATCH), jnp.float32),
    compiler_params=pltpu.CompilerParams(needs_layout_passes=False),
    scratch_types=[
        pltpu.VMEM((_B_PER_W, 128), jnp.float32),
        pltpu.VMEM((EMB, _B_PER_W), jnp.float32),
    ],
)
def _compact(acc_hbm, out_hbm, rb_v, cols_v):
    wid = lax.axis_index("s") * _NC + lax.axis_index("c")
    base = wid * _B_PER_W
    lane = lax.iota(jnp.int32, 16)
    pltpu.sync_copy(acc_hbm.at[pl.ds(base, _B_PER_W)], rb_v)

    def body(u, carry):
        row = plsc.load_gather(rb_v, [jnp.broadcast_to(u, (16,)), lane])
        plsc.store_scatter(cols_v, [lane, jnp.broadcast_to(u, (16,))], row)
        return carry

    lax.fori_loop(0, _B_PER_W, body, 0)
    pltpu.sync_copy(cols_v, out_hbm.at[:, pl.ds(base, _B_PER_W)])


def kernel(user_id, table):
    idx = user_id.astype(jnp.int32)
    acc = _sweep(table.T, idx)
    out_t = _compact(acc)
    return out_t.T[:, None, :]


# speedup vs baseline: 1.3884x; 1.3707x over previous
import functools

import jax
import jax.numpy as jnp
from jax import lax
from jax.experimental import pallas as pl
from jax.experimental.pallas import tpu as pltpu
from jax.experimental.pallas import tpu_sc as plsc

EMB = 16
BATCH = 16384

_info = plsc.get_sparse_core_info()
_NC, _NS = _info.num_cores, _info.num_subcores
_NW = _NC * _NS
_B_PER_W = BATCH // _NW
_WAVE = 16
_N_WAVES = _B_PER_W // _WAVE

_mesh = plsc.VectorSubcoreMesh(core_axis_name="c", subcore_axis_name="s")


@functools.partial(
    pl.kernel,
    mesh=_mesh,
    out_type=jax.ShapeDtypeStruct((EMB, BATCH), jnp.float32),
    compiler_params=pltpu.CompilerParams(needs_layout_passes=False),
    scratch_types=[
        pltpu.VMEM((_B_PER_W,), jnp.int32),
        pltpu.VMEM((3, _WAVE, EMB, 128), jnp.float32),
        pltpu.VMEM((EMB, _B_PER_W), jnp.float32),
        pltpu.SemaphoreType.DMA,
        pltpu.SemaphoreType.DMA,
        pltpu.SemaphoreType.DMA,
    ],
)
def _gather_cols(
    table_hbm, idx_hbm, out_hbm, idx_v, stage_v, cols_v, sem0, sem1, sem2
):
    wid = lax.axis_index("s") * _NC + lax.axis_index("c")
    base = wid * _B_PER_W
    pltpu.sync_copy(idx_hbm.at[pl.ds(base, _B_PER_W)], idx_v)

    lane = lax.iota(jnp.int32, 16)
    sems = (sem0, sem1, sem2)

    def issue(w, buf):
        v = idx_v[pl.ds(w * _WAVE, _WAVE)]
        for u in range(_WAVE):
            c = v[u]
            j = pl.multiple_of((c >> 7) << 7, 128)
            pltpu.async_copy(
                table_hbm.at[:, pl.ds(j, 128)], stage_v.at[buf, u], sems[buf]
            )

    def extract(w, buf):
        v = idx_v[pl.ds(w * _WAVE, _WAVE)]
        for u in range(_WAVE):
            pltpu.make_async_copy(
                table_hbm.at[:, pl.ds(0, 128)], stage_v.at[buf, u], sems[buf]
            ).wait()
        for u in range(_WAVE):
            k = jnp.broadcast_to(v[u] & 127, (16,))
            col = plsc.load_gather(stage_v.at[buf, u], [lane, k])
            plsc.store_scatter(
                cols_v, [lane, jnp.broadcast_to(w * _WAVE + u, (16,))], col
            )

    issue(0, 0)
    issue(1, 1)

    def body(w, carry):
        @pl.when(w % 3 == 0)
        def _():
            issue(w + 2, 2)
            extract(w, 0)

        @pl.when(w % 3 == 1)
        def _():
            issue(w + 2, 0)
            extract(w, 1)

        @pl.when(w % 3 == 2)
        def _():
            issue(w + 2, 1)
            extract(w, 2)

        return carry

    lax.fori_loop(0, _N_WAVES - 2, body, 0)
    extract(_N_WAVES - 2, (_N_WAVES - 2) % 3)
    extract(_N_WAVES - 1, (_N_WAVES - 1) % 3)
    pltpu.sync_copy(cols_v, out_hbm.at[:, pl.ds(base, _B_PER_W)])


def kernel(user_id, table):
    idx = user_id.astype(jnp.int32)
    out_t = _gather_cols(table.T, idx)
    return out_t.T[:, None, :]
